# baseline (speedup 1.0000x reference)
import functools
import math

import jax
import jax.numpy as jnp
import numpy as np
from jax import lax
from jax.experimental import pallas as pl
from jax.experimental.pallas import tpu as pltpu
from jax.experimental.pallas import tpu_sc as plsc

NXV, NYV, NZV = 200, 200, 4
N_VOX = NXV * NYV * NZV
N_XY = NXV * NYV
NCAM = 6
CH_IN, CH_OUT = 256, 80
FH, FW = 64, 176
FHW = FH * FW
ZERO_ROW_BLOCK = NCAM
STRIDE = 4.0
BN_EPS = 1e-5

SC_CORES, SC_SUBCORES, SC_LANES = 2, 16, 16
NW = SC_CORES * SC_SUBCORES
XY_PER_W = N_XY // NW
CHUNK_XY = 25
IDX_PER_CHUNK = CHUNK_XY * NZV
NCHUNK = XY_PER_W // CHUNK_XY


TAB_W = 128


TAB_BLK = FHW // 2


def _table_body(feat_ref, w_ref, out_ref):
    j = pl.program_id(0)
    mm = lax.dot_general(
        feat_ref[0], w_ref[...],
        dimension_numbers=(((0,), (1,)), ((), ())),
        preferred_element_type=jnp.float32,
        precision=lax.Precision.HIGHEST,
    )
    mm = jnp.where(j < NCAM, mm, jnp.zeros_like(mm))
    out_ref[0, :, 0:CH_OUT] = mm
    out_ref[0, :, CH_OUT:TAB_W] = jnp.zeros((TAB_BLK, TAB_W - CH_OUT), jnp.float32)


def _build_table(feat, conv_w):
    return pl.pallas_call(
        _table_body,
        grid=(NCAM + 1, FHW // TAB_BLK),
        in_specs=[
            pl.BlockSpec((1, CH_IN, TAB_BLK),
                         lambda j, p: (jnp.minimum(j, NCAM - 1), 0, p)),
            pl.BlockSpec((CH_OUT, CH_IN), lambda j, p: (0, 0)),
        ],
        out_specs=pl.BlockSpec((1, TAB_BLK, TAB_W), lambda j, p: (j, p, 0)),
        out_shape=jax.ShapeDtypeStruct((NCAM + 1, FHW, TAB_W), jnp.float32),
    )(feat, conv_w)


ADDR_BLK = 20000
ADDR_GRID = N_VOX // ADDR_BLK
_F32 = np.float32
_ORIGIN_X = float(_F32(0.0) - _F32(NXV) / _F32(2.0) * _F32(0.5))
_ORIGIN_Y = float(_F32(0.0) - _F32(NYV) / _F32(2.0) * _F32(0.5))
_ORIGIN_Z = float(_F32(-1.7) - _F32(NZV) / _F32(2.0) * _F32(1.0))


def _addr_body(rm_ref, proj_ref, par_ref, out_ref):
    i = pl.program_id(0)
    col = lax.broadcasted_iota(jnp.int32, (1, ADDR_BLK), 1)
    n = i * ADDR_BLK + col
    ix = n // (NYV * NZV)
    rem = n - ix * (NYV * NZV)
    iy = rem // NZV
    iz = rem - iy * NZV
    px = (ix.astype(jnp.float32) * 0.5 + _ORIGIN_X) - par_ref[0]
    py = (iy.astype(jnp.float32) * 0.5 + _ORIGIN_Y) - par_ref[1]
    pz = (iz.astype(jnp.float32) * 1.0 + _ORIGIN_Z) - par_ref[2]
    p4 = jnp.concatenate([px, py, pz], axis=0)
    q = lax.dot_general(rm_ref[...], p4, (((0,), (0,)), ((), ())),
                        preferred_element_type=jnp.float32)
    pt = jnp.concatenate([q, jnp.ones((1, ADDR_BLK), jnp.float32)], axis=0)
    addr = jnp.full((1, ADDR_BLK), ZERO_ROW_BLOCK * FHW, jnp.int32)
    for j in range(NCAM):
        p2i = lax.dot_general(proj_ref[j], pt, (((1,), (0,)), ((), ())),
                              preferred_element_type=jnp.float32)
        u = p2i[0:1] / p2i[2:3] + par_ref[3 + 2 * j]
        v = p2i[1:2] / p2i[2:3] + par_ref[4 + 2 * j]
        ui = jnp.round(u / STRIDE).astype(jnp.int32)
        vi = jnp.round(v / STRIDE).astype(jnp.int32)
        val = (ui >= 0) & (vi >= 0) & (ui < FW) & (vi < FH) & (p2i[2:3] > 0.0)
        addr = jnp.where(val, (j * FHW + vi * FW) + ui, addr)
    out_ref[0] = addr


def _build_addr(rm, proj, params):
    return pl.pallas_call(
        _addr_body,
        grid=(ADDR_GRID,),
        in_specs=[
            pl.BlockSpec((3, 3), lambda i: (0, 0)),
            pl.BlockSpec((NCAM, 3, 4), lambda i: (0, 0, 0)),
            pl.BlockSpec(memory_space=pltpu.SMEM),
        ],
        out_specs=pl.BlockSpec((1, 1, ADDR_BLK), lambda i: (i, 0, 0)),
        out_shape=jax.ShapeDtypeStruct((ADDR_GRID, 1, ADDR_BLK), jnp.int32),
    )(rm, proj, params)


N_PART = 5
NCHUNK_P = NCHUNK // N_PART
ROWS_P = XY_PER_W // N_PART
NBUF = 5


def _sc_body(table_hbm, addr_hbm, out_hbm, idx_v, rows_v, acc_v, *sems):
    wid = lax.axis_index("s") * SC_CORES + lax.axis_index("c")
    pltpu.sync_copy(addr_hbm.at[wid], idx_v)

    for b in range(NBUF):
        pltpu.async_copy(table_hbm.at[idx_v.at[b]], rows_v.at[b], sems[b])

    def group(g, _):
        for b in range(NBUF):
            k = g * NBUF + b
            pltpu.make_async_copy(
                table_hbm.at[idx_v.at[k]], rows_v.at[b], sems[b]).wait()
            part = k // NCHUNK_P
            kin = k - part * NCHUNK_P
            row0 = kin * CHUNK_XY
            for cc in range(CHUNK_XY):
                for d in range(CH_OUT // SC_LANES):
                    sl = pl.ds(d * SC_LANES, SC_LANES)
                    s = ((rows_v[b, 4 * cc, sl] + rows_v[b, 4 * cc + 1, sl])
                         + (rows_v[b, 4 * cc + 2, sl] + rows_v[b, 4 * cc + 3, sl]))
                    acc_v[row0 + cc, sl] = s

            @pl.when(kin == NCHUNK_P - 1)
            def _():
                pltpu.sync_copy(acc_v, out_hbm.at[wid, part])

            @pl.when(k + NBUF < NCHUNK)
            def _():
                pltpu.async_copy(
                    table_hbm.at[idx_v.at[k + NBUF]], rows_v.at[b], sems[b])
        return 0

    lax.fori_loop(0, NCHUNK // NBUF, group, 0)


def _sc_gather(table, addr):
    call = functools.partial(
        pl.kernel,
        out_type=jax.ShapeDtypeStruct((NW, N_PART, ROWS_P, CH_OUT), jnp.float32),
        mesh=plsc.VectorSubcoreMesh(core_axis_name="c", subcore_axis_name="s"),
        scratch_types=(
            [pltpu.VMEM((NCHUNK, IDX_PER_CHUNK), jnp.int32),
             pltpu.VMEM((NBUF, IDX_PER_CHUNK, TAB_W), jnp.float32),
             pltpu.VMEM((ROWS_P, CH_OUT), jnp.float32)]
            + [pltpu.SemaphoreType.DMA] * NBUF
        ),
    )(_sc_body)
    return call(table, addr).reshape(N_XY, CH_OUT)


STAT_BLK = 5000
N_STAT = N_XY // STAT_BLK


def _stats_body(x_ref, out_ref):
    i = pl.program_id(0)
    x = x_ref[...]
    s = jnp.sum(x, axis=0)
    sq = jnp.sum(x * x, axis=0)

    @pl.when(i == 0)
    def _():
        out_ref[0, :] = s
        out_ref[1, :] = sq

    @pl.when(i > 0)
    def _():
        out_ref[0, :] = out_ref[0, :] + s
        out_ref[1, :] = out_ref[1, :] + sq

    @pl.when(i == N_STAT - 1)
    def _():
        mean = out_ref[0, :] / N_XY
        var = out_ref[1, :] / N_XY - mean * mean
        out_ref[0, :] = mean
        out_ref[1, :] = jnp.sqrt(var + BN_EPS)


def _bn_stats(x):
    return pl.pallas_call(
        _stats_body,
        grid=(N_STAT,),
        in_specs=[pl.BlockSpec((STAT_BLK, CH_OUT), lambda i: (i, 0))],
        out_specs=pl.BlockSpec((2, CH_OUT), lambda i: (0, 0)),
        out_shape=jax.ShapeDtypeStruct((2, CH_OUT), jnp.float32),
    )(x)


N_XY_PAD = 40960
NORM_BLK = 5120
N_NORM = N_XY_PAD // NORM_BLK


def _norm_body(x_ref, st_ref, g_ref, b_ref, out_ref):
    i = pl.program_id(0)
    x = x_ref[...]
    mean = st_ref[0:1, :]
    denom = st_ref[1:2, :]
    xn = (x - mean) / denom * g_ref[...] + b_ref[...]
    xn = jnp.maximum(xn, 0.0)
    r = lax.broadcasted_iota(jnp.int32, (CH_OUT, CH_OUT), 0)
    c = lax.broadcasted_iota(jnp.int32, (CH_OUT, CH_OUT), 1)
    eye = (r == c).astype(jnp.float32)
    y = lax.dot_general(
        eye, xn,
        dimension_numbers=(((1,), (1,)), ((), ())),
        preferred_element_type=jnp.float32,
        precision=lax.Precision.HIGHEST,
    )
    out_ref[:, pl.ds(i * NORM_BLK, NORM_BLK)] = y


def _bn_norm(x_pad, stats, gamma, beta):
    return pl.pallas_call(
        _norm_body,
        grid=(N_NORM,),
        in_specs=[
            pl.BlockSpec((NORM_BLK, CH_OUT), lambda i: (i, 0)),
            pl.BlockSpec((2, CH_OUT), lambda i: (0, 0)),
            pl.BlockSpec((1, CH_OUT), lambda i: (0, 0)),
            pl.BlockSpec((1, CH_OUT), lambda i: (0, 0)),
        ],
        out_specs=pl.BlockSpec((CH_OUT, N_XY_PAD), lambda i: (0, 0)),
        out_shape=jax.ShapeDtypeStruct((CH_OUT, N_XY_PAD), jnp.float32),
    )(x_pad, stats, gamma, beta)


def kernel(mlvl_feats, ori_points, img, lidar2image, img_aug_matrix,
           lidar_aug_matrix, img_metas, conv_w, conv_b, bn_gamma, bn_beta):
    del ori_points, img, img_metas, conv_b
    feat = mlvl_feats[0].reshape(NCAM, CH_IN, FHW)
    table = _build_table(feat, conv_w).reshape((NCAM + 1) * FHW, TAB_W)

    lidar_aug_m = lidar_aug_matrix[0]
    t = lidar_aug_m[:3, -1]
    rm = lidar_aug_m[:3, :3]
    img_aug_m = img_aug_matrix[0]
    tt = img_aug_m[..., -1]
    img_aug_m_r = img_aug_m.at[:, :-1, -1].set(0.0)
    projection = jnp.matmul(img_aug_m_r, lidar2image[0])[:, :3, :]
    params = jnp.concatenate([t.reshape(-1), tt[:, :2].reshape(-1)])

    addr = _build_addr(rm, projection, params).reshape(NW, NCHUNK, IDX_PER_CHUNK)
    bev = _sc_gather(table, addr)
    stats = _bn_stats(bev)
    bev_pad = jnp.concatenate(
        [bev, jnp.zeros((N_XY_PAD - N_XY, CH_OUT), jnp.float32)], axis=0)
    out = _bn_norm(bev_pad, stats, bn_gamma.reshape(1, CH_OUT),
                   bn_beta.reshape(1, CH_OUT))
    return out[:, :N_XY].reshape(1, CH_OUT, NXV, NYV)


# speedup vs baseline: 1.0015x; 1.0015x over previous
import functools
import math

import jax
import jax.numpy as jnp
import numpy as np
from jax import lax
from jax.experimental import pallas as pl
from jax.experimental.pallas import tpu as pltpu
from jax.experimental.pallas import tpu_sc as plsc

NXV, NYV, NZV = 200, 200, 4
N_VOX = NXV * NYV * NZV
N_XY = NXV * NYV
NCAM = 6
CH_IN, CH_OUT = 256, 80
FH, FW = 64, 176
FHW = FH * FW
ZERO_ROW_BLOCK = NCAM
STRIDE = 4.0
BN_EPS = 1e-5

SC_CORES, SC_SUBCORES, SC_LANES = 2, 16, 16
NW = SC_CORES * SC_SUBCORES
XY_PER_W = N_XY // NW
CHUNK_XY = 25
IDX_PER_CHUNK = CHUNK_XY * NZV
NCHUNK = XY_PER_W // CHUNK_XY


TAB_W = 128


TAB_BLK = FHW // 2


def _table_body(feat_ref, w_ref, out_ref):
    j = pl.program_id(0)
    mm = lax.dot_general(
        feat_ref[0], w_ref[...],
        dimension_numbers=(((0,), (1,)), ((), ())),
        preferred_element_type=jnp.float32,
        precision=lax.Precision.HIGHEST,
    )
    mm = jnp.where(j < NCAM, mm, jnp.zeros_like(mm))
    out_ref[0, :, 0:CH_OUT] = mm
    out_ref[0, :, CH_OUT:TAB_W] = jnp.zeros((TAB_BLK, TAB_W - CH_OUT), jnp.float32)


def _build_table(feat, conv_w):
    return pl.pallas_call(
        _table_body,
        grid=(NCAM + 1, FHW // TAB_BLK),
        in_specs=[
            pl.BlockSpec((1, CH_IN, TAB_BLK),
                         lambda j, p: (jnp.minimum(j, NCAM - 1), 0, p)),
            pl.BlockSpec((CH_OUT, CH_IN), lambda j, p: (0, 0)),
        ],
        out_specs=pl.BlockSpec((1, TAB_BLK, TAB_W), lambda j, p: (j, p, 0)),
        out_shape=jax.ShapeDtypeStruct((NCAM + 1, FHW, TAB_W), jnp.float32),
    )(feat, conv_w)


ADDR_BLK = 20000
ADDR_GRID = N_VOX // ADDR_BLK
_F32 = np.float32
_ORIGIN_X = float(_F32(0.0) - _F32(NXV) / _F32(2.0) * _F32(0.5))
_ORIGIN_Y = float(_F32(0.0) - _F32(NYV) / _F32(2.0) * _F32(0.5))
_ORIGIN_Z = float(_F32(-1.7) - _F32(NZV) / _F32(2.0) * _F32(1.0))


def _addr_body(rm_ref, proj_ref, par_ref, out_ref):
    i = pl.program_id(0)
    col = lax.broadcasted_iota(jnp.int32, (1, ADDR_BLK), 1)
    n = i * ADDR_BLK + col
    ix = n // (NYV * NZV)
    rem = n - ix * (NYV * NZV)
    iy = rem // NZV
    iz = rem - iy * NZV
    px = (ix.astype(jnp.float32) * 0.5 + _ORIGIN_X) - par_ref[0]
    py = (iy.astype(jnp.float32) * 0.5 + _ORIGIN_Y) - par_ref[1]
    pz = (iz.astype(jnp.float32) * 1.0 + _ORIGIN_Z) - par_ref[2]
    p4 = jnp.concatenate([px, py, pz], axis=0)
    q = lax.dot_general(rm_ref[...], p4, (((0,), (0,)), ((), ())),
                        preferred_element_type=jnp.float32)
    pt = jnp.concatenate([q, jnp.ones((1, ADDR_BLK), jnp.float32)], axis=0)
    addr = jnp.full((1, ADDR_BLK), ZERO_ROW_BLOCK * FHW, jnp.int32)
    for j in range(NCAM):
        p2i = lax.dot_general(proj_ref[j], pt, (((1,), (0,)), ((), ())),
                              preferred_element_type=jnp.float32)
        u = p2i[0:1] / p2i[2:3] + par_ref[3 + 2 * j]
        v = p2i[1:2] / p2i[2:3] + par_ref[4 + 2 * j]
        ui = jnp.round(u / STRIDE).astype(jnp.int32)
        vi = jnp.round(v / STRIDE).astype(jnp.int32)
        val = (ui >= 0) & (vi >= 0) & (ui < FW) & (vi < FH) & (p2i[2:3] > 0.0)
        addr = jnp.where(val, (j * FHW + vi * FW) + ui, addr)
    out_ref[0] = addr


def _build_addr(rm, proj, params):
    return pl.pallas_call(
        _addr_body,
        grid=(ADDR_GRID,),
        in_specs=[
            pl.BlockSpec((3, 3), lambda i: (0, 0)),
            pl.BlockSpec((NCAM, 3, 4), lambda i: (0, 0, 0)),
            pl.BlockSpec(memory_space=pltpu.SMEM),
        ],
        out_specs=pl.BlockSpec((1, 1, ADDR_BLK), lambda i: (i, 0, 0)),
        out_shape=jax.ShapeDtypeStruct((ADDR_GRID, 1, ADDR_BLK), jnp.int32),
    )(rm, proj, params)


N_PART = 5
NCHUNK_P = NCHUNK // N_PART
ROWS_P = XY_PER_W // N_PART
NBUF = 5


def _sc_body(table_hbm, addr_hbm, out_hbm, idx_v, rows_v, acc_v, *sems):
    wid = lax.axis_index("s") * SC_CORES + lax.axis_index("c")
    pltpu.sync_copy(addr_hbm.at[wid], idx_v)

    for b in range(NBUF):
        pltpu.async_copy(table_hbm.at[idx_v.at[b]], rows_v.at[b], sems[b])

    def group(g, _):
        for b in range(NBUF):
            k = g * NBUF + b
            pltpu.make_async_copy(
                table_hbm.at[idx_v.at[k]], rows_v.at[b], sems[b]).wait()
            part = k // NCHUNK_P
            kin = k - part * NCHUNK_P
            row0 = kin * CHUNK_XY

            def col(cc, _):
                r4 = 4 * cc
                for d in range(CH_OUT // SC_LANES):
                    sl = pl.ds(d * SC_LANES, SC_LANES)
                    s = ((rows_v[b, r4, sl] + rows_v[b, r4 + 1, sl])
                         + (rows_v[b, r4 + 2, sl] + rows_v[b, r4 + 3, sl]))
                    acc_v[row0 + cc, sl] = s
                return 0

            lax.fori_loop(0, CHUNK_XY, col, 0)

            @pl.when(kin == NCHUNK_P - 1)
            def _():
                pltpu.sync_copy(acc_v, out_hbm.at[wid, part])

            @pl.when(k + NBUF < NCHUNK)
            def _():
                pltpu.async_copy(
                    table_hbm.at[idx_v.at[k + NBUF]], rows_v.at[b], sems[b])
        return 0

    lax.fori_loop(0, NCHUNK // NBUF, group, 0)


def _sc_gather(table, addr):
    call = functools.partial(
        pl.kernel,
        out_type=jax.ShapeDtypeStruct((NW, N_PART, ROWS_P, CH_OUT), jnp.float32),
        mesh=plsc.VectorSubcoreMesh(core_axis_name="c", subcore_axis_name="s"),
        scratch_types=(
            [pltpu.VMEM((NCHUNK, IDX_PER_CHUNK), jnp.int32),
             pltpu.VMEM((NBUF, IDX_PER_CHUNK, TAB_W), jnp.float32),
             pltpu.VMEM((ROWS_P, CH_OUT), jnp.float32)]
            + [pltpu.SemaphoreType.DMA] * NBUF
        ),
    )(_sc_body)
    return call(table, addr).reshape(N_XY, CH_OUT)


STAT_BLK = 5000
N_STAT = N_XY // STAT_BLK


def _stats_body(x_ref, out_ref):
    i = pl.program_id(0)
    x = x_ref[...]
    s = jnp.sum(x, axis=0)
    sq = jnp.sum(x * x, axis=0)

    @pl.when(i == 0)
    def _():
        out_ref[0, :] = s
        out_ref[1, :] = sq

    @pl.when(i > 0)
    def _():
        out_ref[0, :] = out_ref[0, :] + s
        out_ref[1, :] = out_ref[1, :] + sq

    @pl.when(i == N_STAT - 1)
    def _():
        mean = out_ref[0, :] / N_XY
        var = out_ref[1, :] / N_XY - mean * mean
        out_ref[0, :] = mean
        out_ref[1, :] = jnp.sqrt(var + BN_EPS)


def _bn_stats(x):
    return pl.pallas_call(
        _stats_body,
        grid=(N_STAT,),
        in_specs=[pl.BlockSpec((STAT_BLK, CH_OUT), lambda i: (i, 0))],
        out_specs=pl.BlockSpec((2, CH_OUT), lambda i: (0, 0)),
        out_shape=jax.ShapeDtypeStruct((2, CH_OUT), jnp.float32),
    )(x)


N_XY_PAD = 40960
NORM_BLK = 5120
N_NORM = N_XY_PAD // NORM_BLK


def _norm_body(x_ref, st_ref, g_ref, b_ref, out_ref):
    i = pl.program_id(0)
    x = x_ref[...]
    mean = st_ref[0:1, :]
    denom = st_ref[1:2, :]
    xn = (x - mean) / denom * g_ref[...] + b_ref[...]
    xn = jnp.maximum(xn, 0.0)
    r = lax.broadcasted_iota(jnp.int32, (CH_OUT, CH_OUT), 0)
    c = lax.broadcasted_iota(jnp.int32, (CH_OUT, CH_OUT), 1)
    eye = (r == c).astype(jnp.float32)
    y = lax.dot_general(
        eye, xn,
        dimension_numbers=(((1,), (1,)), ((), ())),
        preferred_element_type=jnp.float32,
        precision=lax.Precision.HIGHEST,
    )
    out_ref[:, pl.ds(i * NORM_BLK, NORM_BLK)] = y


def _bn_norm(x_pad, stats, gamma, beta):
    return pl.pallas_call(
        _norm_body,
        grid=(N_NORM,),
        in_specs=[
            pl.BlockSpec((NORM_BLK, CH_OUT), lambda i: (i, 0)),
            pl.BlockSpec((2, CH_OUT), lambda i: (0, 0)),
            pl.BlockSpec((1, CH_OUT), lambda i: (0, 0)),
            pl.BlockSpec((1, CH_OUT), lambda i: (0, 0)),
        ],
        out_specs=pl.BlockSpec((CH_OUT, N_XY_PAD), lambda i: (0, 0)),
        out_shape=jax.ShapeDtypeStruct((CH_OUT, N_XY_PAD), jnp.float32),
    )(x_pad, stats, gamma, beta)


def kernel(mlvl_feats, ori_points, img, lidar2image, img_aug_matrix,
           lidar_aug_matrix, img_metas, conv_w, conv_b, bn_gamma, bn_beta):
    del ori_points, img, img_metas, conv_b
    feat = mlvl_feats[0].reshape(NCAM, CH_IN, FHW)
    table = _build_table(feat, conv_w).reshape((NCAM + 1) * FHW, TAB_W)

    lidar_aug_m = lidar_aug_matrix[0]
    t = lidar_aug_m[:3, -1]
    rm = lidar_aug_m[:3, :3]
    img_aug_m = img_aug_matrix[0]
    tt = img_aug_m[..., -1]
    img_aug_m_r = img_aug_m.at[:, :-1, -1].set(0.0)
    projection = jnp.matmul(img_aug_m_r, lidar2image[0])[:, :3, :]
    params = jnp.concatenate([t.reshape(-1), tt[:, :2].reshape(-1)])

    addr = _build_addr(rm, projection, params).reshape(NW, NCHUNK, IDX_PER_CHUNK)
    bev = _sc_gather(table, addr)
    stats = _bn_stats(bev)
    bev_pad = jnp.concatenate(
        [bev, jnp.zeros((N_XY_PAD - N_XY, CH_OUT), jnp.float32)], axis=0)
    out = _bn_norm(bev_pad, stats, bn_gamma.reshape(1, CH_OUT),
                   bn_beta.reshape(1, CH_OUT))
    return out[:, :N_XY].reshape(1, CH_OUT, NXV, NYV)
